# baseline (speedup 1.0000x reference)
import functools

import jax
import jax.numpy as jnp
from jax import lax
from jax.experimental import pallas as pl
from jax.experimental.pallas import tpu as pltpu
from jax.experimental.pallas import tpu_sc as plsc

N_ROWS = 100000
D = 128
N_TABLE = 1024
NC = 2
NS = 16
NW = NC * NS

CHUNK = 3128
LAST_BASE = N_ROWS - CHUNK
BLK = 136
NBLK = CHUNK // BLK
NBUF = 6
LOOK = NBUF - 2


def _body(x_hbm, idx_hbm, b_hbm, out_hbm, idx_all, b_sh, *bufs_and_sems):
    rows = bufs_and_sems[:NBUF]
    sem_x = bufs_and_sems[NBUF:2 * NBUF]
    sem_g = bufs_and_sems[2 * NBUF:3 * NBUF]
    sem_o = bufs_and_sems[3 * NBUF:4 * NBUF]
    sem_i = bufs_and_sems[4 * NBUF]

    sid = lax.axis_index("s")
    wid = sid * NC + lax.axis_index("c")
    base = jnp.minimum(wid * CHUNK, LAST_BASE)

    x_h = [None] * NBLK
    g_h = [None] * NBLK
    o_h = [None] * NBLK

    def start_x(j):
        buf = j % NBUF
        x_h[j] = pltpu.async_copy(
            x_hbm.at[pl.ds(base + j * BLK, BLK)], rows[buf], sem_x[buf])

    def start_gather_add(j):
        buf = j % NBUF
        g_h[j] = pltpu.async_copy(
            b_sh.at[idx_all.at[pl.ds(j * BLK, BLK)]], rows[buf],
            sem_g[buf], add=True)

    def start_out(j):
        buf = j % NBUF
        o_h[j] = pltpu.async_copy(
            rows[buf], out_hbm.at[pl.ds(base + j * BLK, BLK)], sem_o[buf])

    for j in range(min(LOOK, NBLK)):
        start_x(j)
    idx_h = pltpu.async_copy(idx_hbm.at[pl.ds(base, CHUNK)], idx_all, sem_i)

    @pl.when(sid == 0)
    def _():
        pltpu.sync_copy(b_hbm, b_sh)

    idx_h.wait()
    plsc.subcore_barrier()

    for j in range(NBLK):
        x_h[j].wait()
        start_gather_add(j)
        if j + LOOK < NBLK:
            if j >= 2:
                o_h[j - 2].wait()
            start_x(j + LOOK)
        if j >= 1:
            g_h[j - 1].wait()
            start_out(j - 1)
    g_h[NBLK - 1].wait()
    start_out(NBLK - 1)
    for j in range(max(0, NBLK - LOOK - 2), NBLK):
        o_h[j].wait()


@functools.partial(jax.jit, donate_argnums=())
def _run(x, batch, b):
    mesh = plsc.VectorSubcoreMesh(
        core_axis_name="c", subcore_axis_name="s", num_cores=NC, num_subcores=NS
    )
    f = pl.kernel(
        _body,
        out_type=jax.ShapeDtypeStruct((N_ROWS, D), jnp.float32),
        mesh=mesh,
        scratch_types=(
            [pltpu.VMEM((CHUNK,), jnp.int32),
             pltpu.MemorySpace.VMEM_SHARED((N_TABLE, D), jnp.float32)]
            + [pltpu.VMEM((BLK, D), jnp.float32) for _ in range(NBUF)]
            + [pltpu.SemaphoreType.DMA for _ in range(3 * NBUF + 1)]
        ),
    )
    return f(x, batch, b)


def kernel(x, batch, b):
    return _run(x, batch.astype(jnp.int32), b)


# speedup vs baseline: 1.0286x; 1.0286x over previous
import functools

import jax
import jax.numpy as jnp
from jax import lax
from jax.experimental import pallas as pl
from jax.experimental.pallas import tpu as pltpu
from jax.experimental.pallas import tpu_sc as plsc

N_ROWS = 100000
D = 128
N_TABLE = 1024
NC = 2
NS = 16
NW = NC * NS

CHUNK = 3136
LAST_BASE = N_ROWS - CHUNK
BLK = 224
NBLK = CHUNK // BLK
NBUF = 4
LOOK = NBUF - 2


def _body(x_hbm, idx_hbm, b_hbm, out_hbm, idx_all, b_sh, *bufs_and_sems):
    rows = bufs_and_sems[:NBUF]
    sem_x = bufs_and_sems[NBUF:2 * NBUF]
    sem_g = bufs_and_sems[2 * NBUF:3 * NBUF]
    sem_o = bufs_and_sems[3 * NBUF:4 * NBUF]
    sem_i = bufs_and_sems[4 * NBUF]

    sid = lax.axis_index("s")
    wid = sid * NC + lax.axis_index("c")
    base = jnp.minimum(wid * CHUNK, LAST_BASE)

    x_h = [None] * NBLK
    g_h = [None] * NBLK
    o_h = [None] * NBLK

    def start_x(j):
        buf = j % NBUF
        x_h[j] = pltpu.async_copy(
            x_hbm.at[pl.ds(base + j * BLK, BLK)], rows[buf], sem_x[buf])

    def start_gather_add(j):
        buf = j % NBUF
        g_h[j] = pltpu.async_copy(
            b_sh.at[idx_all.at[pl.ds(j * BLK, BLK)]], rows[buf],
            sem_g[buf], add=True)

    def start_out(j):
        buf = j % NBUF
        o_h[j] = pltpu.async_copy(
            rows[buf], out_hbm.at[pl.ds(base + j * BLK, BLK)], sem_o[buf])

    for j in range(min(LOOK, NBLK)):
        start_x(j)
    idx_h = pltpu.async_copy(idx_hbm.at[pl.ds(base, CHUNK)], idx_all, sem_i)

    @pl.when(sid == 0)
    def _():
        pltpu.sync_copy(b_hbm, b_sh)

    idx_h.wait()
    plsc.subcore_barrier()

    for j in range(NBLK):
        x_h[j].wait()
        start_gather_add(j)
        if j + LOOK < NBLK:
            if j >= 2:
                o_h[j - 2].wait()
            start_x(j + LOOK)
        if j >= 1:
            g_h[j - 1].wait()
            start_out(j - 1)
    g_h[NBLK - 1].wait()
    start_out(NBLK - 1)
    for j in range(max(0, NBLK - LOOK - 2), NBLK):
        o_h[j].wait()


@functools.partial(jax.jit, donate_argnums=())
def _run(x, batch, b):
    mesh = plsc.VectorSubcoreMesh(
        core_axis_name="c", subcore_axis_name="s", num_cores=NC, num_subcores=NS
    )
    f = pl.kernel(
        _body,
        out_type=jax.ShapeDtypeStruct((N_ROWS, D), jnp.float32),
        mesh=mesh,
        scratch_types=(
            [pltpu.VMEM((CHUNK,), jnp.int32),
             pltpu.MemorySpace.VMEM_SHARED((N_TABLE, D), jnp.float32)]
            + [pltpu.VMEM((BLK, D), jnp.float32) for _ in range(NBUF)]
            + [pltpu.SemaphoreType.DMA for _ in range(3 * NBUF + 1)]
        ),
    )
    return f(x, batch, b)


def kernel(x, batch, b):
    return _run(x, batch.astype(jnp.int32), b)
